# baseline (speedup 1.0000x reference)
import functools

import jax
import jax.numpy as jnp
from jax import lax
from jax.experimental import pallas as pl
from jax.experimental.pallas import tpu as pltpu
from jax.experimental.pallas import tpu_sc as plsc

B, L, K, H, V = 4, 2000, 30, 128, 4
N = B * L
E = N * K
NLAYERS = 6
LP = 2048
TR = 400
TN = 2000
F = 32
INT_MAX = 2147483647

_NC, _NS = 2, 16
_NW = _NC * _NS
_CH = 128
KA = 8
KB = K - KA
_IDX_PAD = KA * N + _NW * (-(-(KB * N // _CH) // _NW)) * _CH


def _ln(x):
    m = jnp.mean(x, axis=-1, keepdims=True)
    v = jnp.mean((x - m) ** 2, axis=-1, keepdims=True)
    return (x - m) / jnp.sqrt(v + 1e-5)


def _pack_bf16_pair(y):
    lo = lax.bitcast_convert_type(y[:, :H // 2], jnp.int32)
    hi = lax.bitcast_convert_type(y[:, H // 2:], jnp.int32)
    lo = ((lo + 0x7FFF + ((lo >> 16) & 1)) >> 16) & 0xFFFF
    hi = (hi + 0x7FFF + ((hi >> 16) & 1)) & (-65536)
    return lo | hi


def _unpack_bf16_pair(w):
    f_even = lax.bitcast_convert_type(w << 16, jnp.float32)
    f_odd = lax.bitcast_convert_type(w & (-65536), jnp.float32)
    return jnp.concatenate([f_even, f_odd], axis=-1)


def _knn_body(xp_ref, xpt_ref, src_ref, d_ref):
    b = pl.program_id(0)
    i = pl.program_id(1)
    xt = xp_ref[0]
    xa = xpt_ref[0]
    x2t = jnp.sum(xt * xt, axis=1, keepdims=True)
    x2a = jnp.sum(xa * xa, axis=0, keepdims=True)
    d2 = x2t + x2a - 2.0 * jnp.dot(xt, xa, preferred_element_type=jnp.float32)
    col = lax.broadcasted_iota(jnp.int32, (TR, LP), 1)
    row_g = i * TR + lax.broadcasted_iota(jnp.int32, (TR, LP), 0)
    d2 = jnp.maximum(d2, 0.0)
    key = lax.bitcast_convert_type(d2, jnp.int32)
    key = (key & (-2048)) | col
    bad = (col == row_g) | (col >= L)
    key = jnp.where(bad, INT_MAX, key)
    for kk in range(K):
        m = jnp.min(key, axis=1, keepdims=True)
        idx = m & 2047
        d2v = lax.bitcast_convert_type(m & (-2048), jnp.float32)
        src_ref[kk, 0] = idx + b * L
        d_ref[kk, 0] = jnp.sqrt(jnp.maximum(d2v, 1e-8))
        key = jnp.where(col == idx, INT_MAX, key)


def _knn(xp, xpt):
    return pl.pallas_call(
        _knn_body,
        grid=(B, L // TR),
        in_specs=[
            pl.BlockSpec((1, TR, 8), lambda b, i: (b, i, 0)),
            pl.BlockSpec((1, 8, LP), lambda b, i: (b, 0, 0)),
        ],
        out_specs=[
            pl.BlockSpec((K, 1, TR, 1), lambda b, i: (0, b * (L // TR) + i, 0, 0)),
            pl.BlockSpec((K, 1, TR, 1), lambda b, i: (0, b * (L // TR) + i, 0, 0)),
        ],
        out_shape=[
            jax.ShapeDtypeStruct((K, N // TR, TR, 1), jnp.int32),
            jax.ShapeDtypeStruct((K, N // TR, TR, 1), jnp.float32),
        ],
    )(xp, xpt)


def _feat_body(d_ref, we_ref, be_ref, fx_ref):
    d = d_ref[0].reshape(TN, 1)
    cen = lax.broadcasted_iota(jnp.int32, (1, 16), 1).astype(jnp.float32) * (20.0 / 15.0)
    z = (d - cen) * (16.0 / 20.0)
    rbf = jnp.exp(-(z * z))
    x = jnp.dot(rbf, we_ref[...], preferred_element_type=jnp.float32) + be_ref[...]
    m = jnp.mean(x, axis=-1, keepdims=True)
    va = jnp.mean((x - m) ** 2, axis=-1, keepdims=True)
    inv_s = lax.rsqrt(va + 1e-5)
    fx_ref[0] = jnp.concatenate(
        [rbf * inv_s, inv_s, m * inv_s, jnp.zeros((TN, F - 18), jnp.float32)],
        axis=1).astype(jnp.bfloat16)


def _feat(d_sub, w_edge, b_edge):
    g = TN // TR
    return pl.pallas_call(
        _feat_body,
        grid=(K, N // TN),
        in_specs=[
            pl.BlockSpec((1, g, TR, 1), lambda k, j: (k, j, 0, 0)),
            pl.BlockSpec((16, H), lambda k, j: (0, 0)),
            pl.BlockSpec((1, H), lambda k, j: (0, 0)),
        ],
        out_specs=pl.BlockSpec((1, TN, F), lambda k, j: (k, j, 0)),
        out_shape=jax.ShapeDtypeStruct((K, N, F), jnp.bfloat16),
    )(d_sub, w_edge, b_edge)


def _node_body(xp_ref, wn_ref, bn_ref, wc_ref, hv_ref, c_ref):
    hv = _ln(jnp.dot(xp_ref[...], wn_ref[...],
                     preferred_element_type=jnp.float32) + bn_ref[...])
    hv_ref[...] = hv
    c_ref[...] = jnp.dot(hv, wc_ref[...],
                         preferred_element_type=jnp.float32)


def _nodes(xp2, wn8, b_node, w1c0):
    return pl.pallas_call(
        _node_body,
        grid=(N // TN,),
        in_specs=[
            pl.BlockSpec((TN, 8), lambda i: (i, 0)),
            pl.BlockSpec((8, H), lambda i: (0, 0)),
            pl.BlockSpec((1, H), lambda i: (0, 0)),
            pl.BlockSpec((H, H), lambda i: (0, 0)),
        ],
        out_specs=[
            pl.BlockSpec((TN, H), lambda i: (i, 0)),
            pl.BlockSpec((TN, H), lambda i: (i, 0)),
        ],
        out_shape=[
            jax.ShapeDtypeStruct((N, H), jnp.float32),
            jax.ShapeDtypeStruct((N, H), jnp.float32),
        ],
    )(xp2, wn8, b_node, w1c0)


def _sc_gather(table, idx_pad, row0, nrows):
    mesh = plsc.VectorSubcoreMesh(core_axis_name="c", subcore_axis_name="s")
    nchunk = nrows // _CH
    cpw = -(-nchunk // _NW)

    @functools.partial(
        pl.kernel,
        out_type=jax.ShapeDtypeStruct((nrows, H), jnp.float32),
        mesh=mesh,
        scratch_types=[
            pltpu.VMEM((cpw * _CH,), jnp.int32),
            pltpu.VMEM((_CH, H), jnp.float32),
            pltpu.VMEM((_CH, H), jnp.float32),
            pltpu.SemaphoreType.DMA,
            pltpu.SemaphoreType.DMA,
        ],
    )
    def k(table_hbm, idx_hbm, out_hbm, idx_v, rows0, rows1, sem0, sem1):
        wid = lax.axis_index("s") * _NC + lax.axis_index("c")
        pltpu.sync_copy(
            idx_hbm.at[pl.ds(row0 + wid * (cpw * _CH), cpw * _CH)], idx_v)
        bufs = (rows0, rows1)
        sems = (sem0, sem1)

        def start(j, p):
            @pl.when((j < cpw) & (wid * cpw + j < nchunk))
            def _():
                pltpu.async_copy(
                    table_hbm.at[idx_v.at[pl.ds(j * _CH, _CH)]],
                    bufs[p], sems[p])

        def drain_store(j, p):
            @pl.when((j < cpw) & (wid * cpw + j < nchunk))
            def _():
                pltpu.make_async_copy(
                    table_hbm.at[pl.ds(0, _CH)], bufs[p], sems[p]).wait()
                pltpu.sync_copy(
                    bufs[p], out_hbm.at[pl.ds((wid * cpw + j) * _CH, _CH)])

        start(0, 0)

        def body(it, _):
            j = it * 2
            start(j + 1, 1)
            drain_store(j, 0)
            start(j + 2, 0)
            drain_store(j + 1, 1)
            return ()

        lax.fori_loop(0, (cpw + 1) // 2, body, ())

    return k(table, idx_pad)


def _edge_step(fx_ref, cs_ref, hv_ref, we_ref, be_ref, w1a_ref, w1b_ref,
               b1_ref, w2b_ref, b2_ref, a_s, mx_s, kk):
    @pl.when(kk == 0)
    def _():
        w1a = w1a_ref[...]
        mx_s[...] = jnp.concatenate(
            [jnp.dot(we_ref[...], w1a, preferred_element_type=jnp.float32),
             jnp.dot(be_ref[...], w1a, preferred_element_type=jnp.float32),
             -jnp.sum(w1a, axis=0, keepdims=True),
             jnp.zeros((F - 18, H), jnp.float32)], axis=0)
        a_s[...] = jnp.dot(hv_ref[...], w1b_ref[...],
                           preferred_element_type=jnp.float32) + b1_ref[...]

    x = jnp.dot(fx_ref[0], mx_s[...].astype(jnp.bfloat16),
                preferred_element_type=jnp.float32)
    x = jnp.maximum(x + cs_ref[0] + a_s[...], 0.0)
    return jnp.maximum(
        jnp.dot(x.astype(jnp.bfloat16), w2b_ref[...],
                preferred_element_type=jnp.float32) + b2_ref[...], 0.0)


def _layer_a_body(fx_ref, cs_ref, hv_ref, we_ref, be_ref, w1a_ref, w1b_ref,
                  b1_ref, w2b_ref, b2_ref, acc_ref, a_s, acc_s, mx_s):
    kk = pl.program_id(1)
    x = _edge_step(fx_ref, cs_ref, hv_ref, we_ref, be_ref, w1a_ref, w1b_ref,
                   b1_ref, w2b_ref, b2_ref, a_s, mx_s, kk)

    @pl.when(kk == 0)
    def _():
        acc_s[...] = x

    @pl.when(kk > 0)
    def _():
        acc_s[...] = acc_s[...] + x

    @pl.when(kk == KA - 1)
    def _():
        acc_ref[...] = acc_s[...]


def _layer_b_body(fx_ref, cs_ref, hv_ref, we_ref, be_ref, w1a_ref, w1b_ref,
                  b1_ref, w2b_ref, b2_ref, acca_ref, w3_ref, b3_ref, wf1_ref,
                  bf1_ref, wf2_ref, bf2_ref, wcn_ref, hvn_ref, cn_ref, a_s,
                  acc_s, mx_s):
    kk = pl.program_id(1)
    x = _edge_step(fx_ref, cs_ref, hv_ref, we_ref, be_ref, w1a_ref, w1b_ref,
                   b1_ref, w2b_ref, b2_ref, a_s, mx_s, kk)

    @pl.when(kk == 0)
    def _():
        acc_s[...] = x

    @pl.when(kk > 0)
    def _():
        acc_s[...] = acc_s[...] + x

    @pl.when(kk == KB - 1)
    def _():
        agg = jnp.dot((acc_s[...] + acca_ref[...]) * (1.0 / K), w3_ref[...],
                      preferred_element_type=jnp.float32) + b3_ref[...]
        hv = _ln(hv_ref[...] + agg)
        ff = jnp.dot(jnp.maximum(jnp.dot(hv, wf1_ref[...],
                                         preferred_element_type=jnp.float32)
                                 + bf1_ref[...], 0.0),
                     wf2_ref[...], preferred_element_type=jnp.float32)
        hv2 = _ln(hv + ff + bf2_ref[...])
        hvn_ref[...] = hv2
        cn_ref[...] = jnp.dot(hv2, wcn_ref[...],
                              preferred_element_type=jnp.float32)


_WSPEC = pl.BlockSpec((H, H), lambda i, k: (0, 0))
_BSPEC = pl.BlockSpec((1, H), lambda i, k: (0, 0))
_SHARED_SPECS = [
    pl.BlockSpec((1, TN, H), lambda i, k: (k, i, 0)),
    pl.BlockSpec((TN, H), lambda i, k: (i, 0)),
    pl.BlockSpec((16, H), lambda i, k: (0, 0)),
    _BSPEC,
    _WSPEC, _WSPEC, _BSPEC,
    _WSPEC, _BSPEC,
]


def _layer_a(fx_km, cs_km, hv, w_edge, b_edge, w1a, w1b, b1, w2, b2):
    return pl.pallas_call(
        _layer_a_body,
        grid=(N // TN, KA),
        in_specs=[pl.BlockSpec((1, TN, F), lambda i, k: (k, i, 0))]
        + _SHARED_SPECS,
        out_specs=pl.BlockSpec((TN, H), lambda i, k: (i, 0)),
        out_shape=jax.ShapeDtypeStruct((N, H), jnp.float32),
        scratch_shapes=[
            pltpu.VMEM((TN, H), jnp.float32),
            pltpu.VMEM((TN, H), jnp.float32),
            pltpu.VMEM((F, H), jnp.float32),
        ],
    )(fx_km, cs_km, hv, w_edge, b_edge, w1a, w1b, b1, w2, b2)


def _layer_b(fx_km, cs_km, hv, w_edge, b_edge, w1a, w1b, b1, w2, b2, acca,
             w3, b3, wf1, bf1, wf2, bf2, wcn):
    return pl.pallas_call(
        _layer_b_body,
        grid=(N // TN, KB),
        in_specs=[pl.BlockSpec((1, TN, F), lambda i, k: (k + KA, i, 0))]
        + _SHARED_SPECS + [
            pl.BlockSpec((TN, H), lambda i, k: (i, 0)),
            _WSPEC, _BSPEC,
            pl.BlockSpec((H, 2 * H), lambda i, k: (0, 0)),
            pl.BlockSpec((1, 2 * H), lambda i, k: (0, 0)),
            pl.BlockSpec((2 * H, H), lambda i, k: (0, 0)),
            _BSPEC,
            _WSPEC,
        ],
        out_specs=[
            pl.BlockSpec((TN, H), lambda i, k: (i, 0)),
            pl.BlockSpec((TN, H), lambda i, k: (i, 0)),
        ],
        out_shape=[
            jax.ShapeDtypeStruct((N, H), jnp.float32),
            jax.ShapeDtypeStruct((N, H), jnp.float32),
        ],
        scratch_shapes=[
            pltpu.VMEM((TN, H), jnp.float32),
            pltpu.VMEM((TN, H), jnp.float32),
            pltpu.VMEM((F, H), jnp.float32),
        ],
    )(fx_km, cs_km, hv, w_edge, b_edge, w1a, w1b, b1, w2, b2, acca, w3, b3,
      wf1, bf1, wf2, bf2, wcn)


def _head_body(hv_ref, wo_ref, bo_ref, wp1_ref, wp2_ref, bp2_ref,
               logits_ref, prjs_ref, pool_s):
    i = pl.program_id(0)
    hv = hv_ref[...]
    logits_ref[...] = jnp.dot(hv, wo_ref[...],
                              preferred_element_type=jnp.float32) + bo_ref[...]
    s = jnp.sum(hv, axis=0, keepdims=True)

    pool_s[i, :] = s[0]

    @pl.when(i == N // TN - 1)
    def _():
        ge = pool_s[...] * (1.0 / L)
        t = jnp.maximum(jnp.dot(ge, wp1_ref[...],
                                preferred_element_type=jnp.float32), 0.0)
        prjs_ref[...] = jnp.dot(t, wp2_ref[...],
                                preferred_element_type=jnp.float32) + bp2_ref[...]


def _head(hv, wo, bo, wp1, wp2, bp2):
    return pl.pallas_call(
        _head_body,
        grid=(N // TN,),
        in_specs=[
            pl.BlockSpec((TN, H), lambda i: (i, 0)),
            pl.BlockSpec((H, V), lambda i: (0, 0)),
            pl.BlockSpec((1, V), lambda i: (0, 0)),
            pl.BlockSpec((H, H), lambda i: (0, 0)),
            pl.BlockSpec((H, H), lambda i: (0, 0)),
            pl.BlockSpec((1, H), lambda i: (0, 0)),
        ],
        out_specs=[
            pl.BlockSpec((TN, V), lambda i: (i, 0)),
            pl.BlockSpec((B, H), lambda i: (0, 0)),
        ],
        out_shape=[
            jax.ShapeDtypeStruct((N, V), jnp.float32),
            jax.ShapeDtypeStruct((B, H), jnp.float32),
        ],
        scratch_shapes=[pltpu.VMEM((B, H), jnp.float32)],
    )(hv, wo, bo, wp1, wp2, bp2)


def kernel(X, S, mask, W_node, b_node, W_edge, b_edge, L_W1, L_b1, L_W2,
           L_b2, L_W3, L_b3, Wff1, bff1, Wff2, bff2, Wp1, Wp2, bp2, Wo, bo):
    xp = jnp.pad(X, ((0, 0), (0, 0), (0, 5)))
    xpt = jnp.pad(jnp.swapaxes(X, 1, 2), ((0, 0), (0, 5), (0, LP - L)))
    src_sub, d_sub = _knn(xp, xpt)

    fx_km = _feat(d_sub, W_edge, b_edge[None, :])

    perm = jnp.concatenate([jnp.arange(0, H, 2), jnp.arange(1, H, 2)])

    wn8 = jnp.pad(W_node, ((0, 5), (0, 0)))
    hv, c = _nodes(xp.reshape(N, 8), wn8, b_node[None, :],
                   L_W1[0, 2 * H:3 * H, :][:, perm])

    idx_pad = jnp.pad(src_sub.reshape(-1), (0, _IDX_PAD - E))

    for l in range(NLAYERS):
        wcn = L_W1[l + 1, 2 * H:3 * H, :] if l < NLAYERS - 1 else L_W1[0, 2 * H:3 * H, :]
        w1a = L_W1[l, 0:H, :][:, perm]
        w1b = L_W1[l, H:2 * H, :][:, perm]
        b1 = L_b1[l][None, perm]
        w2 = L_W2[l][perm, :].astype(jnp.bfloat16)
        b2 = L_b2[l][None, :]
        cs_a = _sc_gather(c, idx_pad, 0, KA * N).reshape(KA, N, H)
        cs_b = _sc_gather(c, idx_pad, KA * N, KB * N).reshape(KB, N, H)
        acca = _layer_a(fx_km, cs_a, hv, W_edge, b_edge[None, :],
                        w1a, w1b, b1, w2, b2)
        hv, c = _layer_b(
            fx_km, cs_b, hv, W_edge, b_edge[None, :],
            w1a, w1b, b1, w2, b2, acca,
            L_W3[l], L_b3[l][None, :],
            Wff1[l], bff1[l][None, :], Wff2[l], bff2[l][None, :],
            wcn[:, perm])

    logits, prjs = _head(hv, Wo, bo[None, :], Wp1, Wp2, bp2[None, :])
    return (logits, S.reshape(-1), prjs)


# speedup vs baseline: 1.0094x; 1.0094x over previous
import functools

import jax
import jax.numpy as jnp
from jax import lax
from jax.experimental import pallas as pl
from jax.experimental.pallas import tpu as pltpu
from jax.experimental.pallas import tpu_sc as plsc

B, L, K, H, V = 4, 2000, 30, 128, 4
N = B * L
E = N * K
NLAYERS = 6
LP = 2048
TR = 400
TN = 2000
F = 32
INT_MAX = 2147483647

_NC, _NS = 2, 16
_NW = _NC * _NS
_CH = 128
KA = 10
KB = K - KA
_IDX_PAD = KA * N + _NW * (-(-(KB * N // _CH) // _NW)) * _CH


def _ln(x):
    m = jnp.mean(x, axis=-1, keepdims=True)
    v = jnp.mean((x - m) ** 2, axis=-1, keepdims=True)
    return (x - m) / jnp.sqrt(v + 1e-5)


def _knn_body(xp_ref, xpt_ref, src_ref, d_ref):
    b = pl.program_id(0)
    i = pl.program_id(1)
    xt = xp_ref[0]
    xa = xpt_ref[0]
    x2t = jnp.sum(xt * xt, axis=1, keepdims=True)
    x2a = jnp.sum(xa * xa, axis=0, keepdims=True)
    d2 = x2t + x2a - 2.0 * jnp.dot(xt, xa, preferred_element_type=jnp.float32)
    col = lax.broadcasted_iota(jnp.int32, (TR, LP), 1)
    row_g = i * TR + lax.broadcasted_iota(jnp.int32, (TR, LP), 0)
    d2 = jnp.maximum(d2, 0.0)
    key = lax.bitcast_convert_type(d2, jnp.int32)
    key = (key & (-2048)) | col
    bad = (col == row_g) | (col >= L)
    key = jnp.where(bad, INT_MAX, key)
    for kk in range(K):
        m = jnp.min(key, axis=1, keepdims=True)
        idx = m & 2047
        d2v = lax.bitcast_convert_type(m & (-2048), jnp.float32)
        src_ref[kk, 0] = idx + b * L
        d_ref[kk, 0] = jnp.sqrt(jnp.maximum(d2v, 1e-8))
        key = jnp.where(col == idx, INT_MAX, key)


def _knn(xp, xpt):
    return pl.pallas_call(
        _knn_body,
        grid=(B, L // TR),
        in_specs=[
            pl.BlockSpec((1, TR, 8), lambda b, i: (b, i, 0)),
            pl.BlockSpec((1, 8, LP), lambda b, i: (b, 0, 0)),
        ],
        out_specs=[
            pl.BlockSpec((K, 1, TR, 1), lambda b, i: (0, b * (L // TR) + i, 0, 0)),
            pl.BlockSpec((K, 1, TR, 1), lambda b, i: (0, b * (L // TR) + i, 0, 0)),
        ],
        out_shape=[
            jax.ShapeDtypeStruct((K, N // TR, TR, 1), jnp.int32),
            jax.ShapeDtypeStruct((K, N // TR, TR, 1), jnp.float32),
        ],
    )(xp, xpt)


def _feat_body(d_ref, we_ref, be_ref, fx_ref):
    d = d_ref[0].reshape(TN, 1)
    cen = lax.broadcasted_iota(jnp.int32, (1, 16), 1).astype(jnp.float32) * (20.0 / 15.0)
    z = (d - cen) * (16.0 / 20.0)
    rbf = jnp.exp(-(z * z))
    x = jnp.dot(rbf, we_ref[...], preferred_element_type=jnp.float32) + be_ref[...]
    m = jnp.mean(x, axis=-1, keepdims=True)
    va = jnp.mean((x - m) ** 2, axis=-1, keepdims=True)
    inv_s = lax.rsqrt(va + 1e-5)
    fx_ref[0] = jnp.concatenate(
        [rbf * inv_s, inv_s, m * inv_s, jnp.zeros((TN, F - 18), jnp.float32)],
        axis=1).astype(jnp.bfloat16)


def _feat(d_sub, w_edge, b_edge):
    g = TN // TR
    return pl.pallas_call(
        _feat_body,
        grid=(K, N // TN),
        in_specs=[
            pl.BlockSpec((1, g, TR, 1), lambda k, j: (k, j, 0, 0)),
            pl.BlockSpec((16, H), lambda k, j: (0, 0)),
            pl.BlockSpec((1, H), lambda k, j: (0, 0)),
        ],
        out_specs=pl.BlockSpec((1, TN, F), lambda k, j: (k, j, 0)),
        out_shape=jax.ShapeDtypeStruct((K, N, F), jnp.bfloat16),
    )(d_sub, w_edge, b_edge)


def _node_body(xp_ref, wn_ref, bn_ref, wc_ref, hv_ref, c_ref):
    hv = _ln(jnp.dot(xp_ref[...], wn_ref[...],
                     preferred_element_type=jnp.float32) + bn_ref[...])
    hv_ref[...] = hv
    c_ref[...] = jnp.dot(hv, wc_ref[...],
                         preferred_element_type=jnp.float32)


def _nodes(xp2, wn8, b_node, w1c0):
    return pl.pallas_call(
        _node_body,
        grid=(N // TN,),
        in_specs=[
            pl.BlockSpec((TN, 8), lambda i: (i, 0)),
            pl.BlockSpec((8, H), lambda i: (0, 0)),
            pl.BlockSpec((1, H), lambda i: (0, 0)),
            pl.BlockSpec((H, H), lambda i: (0, 0)),
        ],
        out_specs=[
            pl.BlockSpec((TN, H), lambda i: (i, 0)),
            pl.BlockSpec((TN, H), lambda i: (i, 0)),
        ],
        out_shape=[
            jax.ShapeDtypeStruct((N, H), jnp.float32),
            jax.ShapeDtypeStruct((N, H), jnp.float32),
        ],
    )(xp2, wn8, b_node, w1c0)


def _sc_gather(table, idx_pad, row0, nrows):
    mesh = plsc.VectorSubcoreMesh(core_axis_name="c", subcore_axis_name="s")
    nchunk = nrows // _CH
    cpw = -(-nchunk // _NW)

    @functools.partial(
        pl.kernel,
        out_type=jax.ShapeDtypeStruct((nrows, H), jnp.float32),
        mesh=mesh,
        scratch_types=[
            pltpu.VMEM((cpw * _CH,), jnp.int32),
            pltpu.VMEM((_CH, H), jnp.float32),
            pltpu.VMEM((_CH, H), jnp.float32),
            pltpu.SemaphoreType.DMA,
            pltpu.SemaphoreType.DMA,
        ],
    )
    def k(table_hbm, idx_hbm, out_hbm, idx_v, rows0, rows1, sem0, sem1):
        wid = lax.axis_index("s") * _NC + lax.axis_index("c")
        pltpu.sync_copy(
            idx_hbm.at[pl.ds(row0 + wid * (cpw * _CH), cpw * _CH)], idx_v)
        bufs = (rows0, rows1)
        sems = (sem0, sem1)

        def start(j, p):
            @pl.when((j < cpw) & (wid * cpw + j < nchunk))
            def _():
                pltpu.async_copy(
                    table_hbm.at[idx_v.at[pl.ds(j * _CH, _CH)]],
                    bufs[p], sems[p])

        def drain_store(j, p):
            @pl.when((j < cpw) & (wid * cpw + j < nchunk))
            def _():
                pltpu.make_async_copy(
                    table_hbm.at[pl.ds(0, _CH)], bufs[p], sems[p]).wait()
                pltpu.sync_copy(
                    bufs[p], out_hbm.at[pl.ds((wid * cpw + j) * _CH, _CH)])

        start(0, 0)

        def body(it, _):
            j = it * 2
            start(j + 1, 1)
            drain_store(j, 0)
            start(j + 2, 0)
            drain_store(j + 1, 1)
            return ()

        lax.fori_loop(0, (cpw + 1) // 2, body, ())

    return k(table, idx_pad)


def _edge_step(fx_ref, cs_ref, hv_ref, we_ref, be_ref, w1a_ref, w1b_ref,
               b1_ref, w2b_ref, b2_ref, a_s, mx_s, kk):
    @pl.when(kk == 0)
    def _():
        w1a = w1a_ref[...]
        mx_s[...] = jnp.concatenate(
            [jnp.dot(we_ref[...], w1a, preferred_element_type=jnp.float32),
             jnp.dot(be_ref[...], w1a, preferred_element_type=jnp.float32),
             -jnp.sum(w1a, axis=0, keepdims=True),
             jnp.zeros((F - 18, H), jnp.float32)], axis=0)
        a_s[...] = jnp.dot(hv_ref[...], w1b_ref[...],
                           preferred_element_type=jnp.float32) + b1_ref[...]

    x = jnp.dot(fx_ref[0], mx_s[...].astype(jnp.bfloat16),
                preferred_element_type=jnp.float32)
    x = jnp.maximum(x + cs_ref[0] + a_s[...], 0.0)
    return jnp.maximum(
        jnp.dot(x.astype(jnp.bfloat16), w2b_ref[...],
                preferred_element_type=jnp.float32) + b2_ref[...], 0.0)


def _layer_a_body(fx_ref, cs_ref, hv_ref, we_ref, be_ref, w1a_ref, w1b_ref,
                  b1_ref, w2b_ref, b2_ref, acc_ref, a_s, acc_s, mx_s):
    kk = pl.program_id(1)
    x = _edge_step(fx_ref, cs_ref, hv_ref, we_ref, be_ref, w1a_ref, w1b_ref,
                   b1_ref, w2b_ref, b2_ref, a_s, mx_s, kk)

    @pl.when(kk == 0)
    def _():
        acc_s[...] = x

    @pl.when(kk > 0)
    def _():
        acc_s[...] = acc_s[...] + x

    @pl.when(kk == KA - 1)
    def _():
        acc_ref[...] = acc_s[...]


def _layer_b_body(fx_ref, cs_ref, hv_ref, we_ref, be_ref, w1a_ref, w1b_ref,
                  b1_ref, w2b_ref, b2_ref, acca_ref, w3_ref, b3_ref, wf1_ref,
                  bf1_ref, wf2_ref, bf2_ref, wcn_ref, hvn_ref, cn_ref, a_s,
                  acc_s, mx_s):
    kk = pl.program_id(1)
    x = _edge_step(fx_ref, cs_ref, hv_ref, we_ref, be_ref, w1a_ref, w1b_ref,
                   b1_ref, w2b_ref, b2_ref, a_s, mx_s, kk)

    @pl.when(kk == 0)
    def _():
        acc_s[...] = x

    @pl.when(kk > 0)
    def _():
        acc_s[...] = acc_s[...] + x

    @pl.when(kk == KB - 1)
    def _():
        agg = jnp.dot((acc_s[...] + acca_ref[...]) * (1.0 / K), w3_ref[...],
                      preferred_element_type=jnp.float32) + b3_ref[...]
        hv = _ln(hv_ref[...] + agg)
        ff = jnp.dot(jnp.maximum(jnp.dot(hv, wf1_ref[...],
                                         preferred_element_type=jnp.float32)
                                 + bf1_ref[...], 0.0),
                     wf2_ref[...], preferred_element_type=jnp.float32)
        hv2 = _ln(hv + ff + bf2_ref[...])
        hvn_ref[...] = hv2
        cn_ref[...] = jnp.dot(hv2, wcn_ref[...],
                              preferred_element_type=jnp.float32)


_WSPEC = pl.BlockSpec((H, H), lambda i, k: (0, 0))
_BSPEC = pl.BlockSpec((1, H), lambda i, k: (0, 0))
_SHARED_SPECS = [
    pl.BlockSpec((1, TN, H), lambda i, k: (k, i, 0)),
    pl.BlockSpec((TN, H), lambda i, k: (i, 0)),
    pl.BlockSpec((16, H), lambda i, k: (0, 0)),
    _BSPEC,
    _WSPEC, _WSPEC, _BSPEC,
    _WSPEC, _BSPEC,
]


def _layer_a(fx_km, cs_km, hv, w_edge, b_edge, w1a, w1b, b1, w2, b2):
    return pl.pallas_call(
        _layer_a_body,
        grid=(N // TN, KA),
        in_specs=[pl.BlockSpec((1, TN, F), lambda i, k: (k, i, 0))]
        + _SHARED_SPECS,
        out_specs=pl.BlockSpec((TN, H), lambda i, k: (i, 0)),
        out_shape=jax.ShapeDtypeStruct((N, H), jnp.float32),
        scratch_shapes=[
            pltpu.VMEM((TN, H), jnp.float32),
            pltpu.VMEM((TN, H), jnp.float32),
            pltpu.VMEM((F, H), jnp.float32),
        ],
    )(fx_km, cs_km, hv, w_edge, b_edge, w1a, w1b, b1, w2, b2)


def _layer_b(fx_km, cs_km, hv, w_edge, b_edge, w1a, w1b, b1, w2, b2, acca,
             w3, b3, wf1, bf1, wf2, bf2, wcn):
    return pl.pallas_call(
        _layer_b_body,
        grid=(N // TN, KB),
        in_specs=[pl.BlockSpec((1, TN, F), lambda i, k: (k + KA, i, 0))]
        + _SHARED_SPECS + [
            pl.BlockSpec((TN, H), lambda i, k: (i, 0)),
            _WSPEC, _BSPEC,
            pl.BlockSpec((H, 2 * H), lambda i, k: (0, 0)),
            pl.BlockSpec((1, 2 * H), lambda i, k: (0, 0)),
            pl.BlockSpec((2 * H, H), lambda i, k: (0, 0)),
            _BSPEC,
            _WSPEC,
        ],
        out_specs=[
            pl.BlockSpec((TN, H), lambda i, k: (i, 0)),
            pl.BlockSpec((TN, H), lambda i, k: (i, 0)),
        ],
        out_shape=[
            jax.ShapeDtypeStruct((N, H), jnp.float32),
            jax.ShapeDtypeStruct((N, H), jnp.float32),
        ],
        scratch_shapes=[
            pltpu.VMEM((TN, H), jnp.float32),
            pltpu.VMEM((TN, H), jnp.float32),
            pltpu.VMEM((F, H), jnp.float32),
        ],
    )(fx_km, cs_km, hv, w_edge, b_edge, w1a, w1b, b1, w2, b2, acca, w3, b3,
      wf1, bf1, wf2, bf2, wcn)


def _head_body(hv_ref, wo_ref, bo_ref, wp1_ref, wp2_ref, bp2_ref,
               logits_ref, prjs_ref, pool_s):
    i = pl.program_id(0)
    hv = hv_ref[...]
    logits_ref[...] = jnp.dot(hv, wo_ref[...],
                              preferred_element_type=jnp.float32) + bo_ref[...]
    s = jnp.sum(hv, axis=0, keepdims=True)

    pool_s[i, :] = s[0]

    @pl.when(i == N // TN - 1)
    def _():
        ge = pool_s[...] * (1.0 / L)
        t = jnp.maximum(jnp.dot(ge, wp1_ref[...],
                                preferred_element_type=jnp.float32), 0.0)
        prjs_ref[...] = jnp.dot(t, wp2_ref[...],
                                preferred_element_type=jnp.float32) + bp2_ref[...]


def _head(hv, wo, bo, wp1, wp2, bp2):
    return pl.pallas_call(
        _head_body,
        grid=(N // TN,),
        in_specs=[
            pl.BlockSpec((TN, H), lambda i: (i, 0)),
            pl.BlockSpec((H, V), lambda i: (0, 0)),
            pl.BlockSpec((1, V), lambda i: (0, 0)),
            pl.BlockSpec((H, H), lambda i: (0, 0)),
            pl.BlockSpec((H, H), lambda i: (0, 0)),
            pl.BlockSpec((1, H), lambda i: (0, 0)),
        ],
        out_specs=[
            pl.BlockSpec((TN, V), lambda i: (i, 0)),
            pl.BlockSpec((B, H), lambda i: (0, 0)),
        ],
        out_shape=[
            jax.ShapeDtypeStruct((N, V), jnp.float32),
            jax.ShapeDtypeStruct((B, H), jnp.float32),
        ],
        scratch_shapes=[pltpu.VMEM((B, H), jnp.float32)],
    )(hv, wo, bo, wp1, wp2, bp2)


def kernel(X, S, mask, W_node, b_node, W_edge, b_edge, L_W1, L_b1, L_W2,
           L_b2, L_W3, L_b3, Wff1, bff1, Wff2, bff2, Wp1, Wp2, bp2, Wo, bo):
    xp = jnp.pad(X, ((0, 0), (0, 0), (0, 5)))
    xpt = jnp.pad(jnp.swapaxes(X, 1, 2), ((0, 0), (0, 5), (0, LP - L)))
    src_sub, d_sub = _knn(xp, xpt)

    fx_km = _feat(d_sub, W_edge, b_edge[None, :])

    perm = jnp.concatenate([jnp.arange(0, H, 2), jnp.arange(1, H, 2)])

    wn8 = jnp.pad(W_node, ((0, 5), (0, 0)))
    hv, c = _nodes(xp.reshape(N, 8), wn8, b_node[None, :],
                   L_W1[0, 2 * H:3 * H, :][:, perm])

    idx_pad = jnp.pad(src_sub.reshape(-1), (0, _IDX_PAD - E))

    for l in range(NLAYERS):
        wcn = L_W1[l + 1, 2 * H:3 * H, :] if l < NLAYERS - 1 else L_W1[0, 2 * H:3 * H, :]
        w1a = L_W1[l, 0:H, :][:, perm]
        w1b = L_W1[l, H:2 * H, :][:, perm]
        b1 = L_b1[l][None, perm]
        w2 = L_W2[l][perm, :].astype(jnp.bfloat16)
        b2 = L_b2[l][None, :]
        cs_a = _sc_gather(c, idx_pad, 0, KA * N).reshape(KA, N, H)
        cs_b = _sc_gather(c, idx_pad, KA * N, KB * N).reshape(KB, N, H)
        acca = _layer_a(fx_km, cs_a, hv, W_edge, b_edge[None, :],
                        w1a, w1b, b1, w2, b2)
        hv, c = _layer_b(
            fx_km, cs_b, hv, W_edge, b_edge[None, :],
            w1a, w1b, b1, w2, b2, acca,
            L_W3[l], L_b3[l][None, :],
            Wff1[l], bff1[l][None, :], Wff2[l], bff2[l][None, :],
            wcn[:, perm])

    logits, prjs = _head(hv, Wo, bo[None, :], Wp1, Wp2, bp2[None, :])
    return (logits, S.reshape(-1), prjs)
